# baseline (speedup 1.0000x reference)
import functools

import jax
import jax.numpy as jnp
from jax import lax
from jax.experimental import pallas as pl
from jax.experimental.pallas import tpu as pltpu
from jax.experimental.pallas import tpu_sc as plsc

_T, _N = 2048, 32768
_BT = 64
_NW = 32
_T_SC = 512
_RPW = _T_SC // _NW
_T_TC = _T - _T_SC
_L = 16


def _tc_body(w_ref, x_ref, o_ref, sw_ref):
    i = pl.program_id(0)

    @pl.when(i == 0)
    def _():
        wv = w_ref[...]
        m = jnp.max(wv)
        e = jnp.exp(wv - m)
        sw_ref[...] = e / jnp.sum(e)

    o_ref[...] = jnp.sum(x_ref[...] * sw_ref[...], axis=1, keepdims=True)


_sc_mesh = plsc.VectorSubcoreMesh(core_axis_name="c", subcore_axis_name="s")


_C = 512


@functools.partial(
    pl.kernel,
    mesh=_sc_mesh,
    out_type=jax.ShapeDtypeStruct((_T_SC,), jnp.float32),
    scratch_types=[
        pltpu.VMEM((_N,), jnp.float32),
        pltpu.VMEM((_RPW, _C), jnp.float32),
        pltpu.VMEM((_RPW, _C), jnp.float32),
        pltpu.VMEM((_RPW,), jnp.float32),
        pltpu.SemaphoreType.DMA,
        pltpu.SemaphoreType.DMA,
    ],
)
def _sc_matvec(x_hbm, w_hbm, o_hbm, ebuf, xblk0, xblk1, ybuf, sem0, sem1):
    wid = lax.axis_index("s") * 2 + lax.axis_index("c")
    nchunks = _N // _L
    lane = lax.iota(jnp.int32, _L)

    def allreduce(v, op):
        for s in (1, 2, 4, 8):
            v = op(v, v.at[lane ^ s].get(mode="promise_in_bounds"))
        return v

    pltpu.sync_copy(w_hbm, ebuf)

    def mx_body(i, acc):
        return jnp.maximum(acc, ebuf[pl.ds(i * _L, _L)])

    m16 = allreduce(
        lax.fori_loop(0, nchunks, mx_body,
                      jnp.full((_L,), -jnp.inf, jnp.float32)),
        jnp.maximum)

    def ex_body(i, s):
        v = jnp.exp(ebuf[pl.ds(i * _L, _L)] - m16)
        ebuf[pl.ds(i * _L, _L)] = v
        return s + v

    s16 = lax.fori_loop(0, nchunks, ex_body, jnp.zeros((_L,), jnp.float32))
    inv_total = 1.0 / allreduce(s16, jnp.add)

    row0 = _T_TC + wid * _RPW
    npairs = _N // (2 * _C)

    def _start(ci, blk, sem):
        pltpu.make_async_copy(
            x_hbm.at[pl.ds(row0, _RPW), pl.ds(ci * _C, _C)], blk, sem
        ).start()

    def _wait(blk, sem):
        pltpu.make_async_copy(
            x_hbm.at[pl.ds(row0, _RPW), pl.ds(0, _C)], blk, sem
        ).wait()

    def _accumulate(blk, cbase, accs):
        for j in range(_C // _L):
            ev = ebuf[pl.ds(cbase + j * _L, _L)]
            for r in range(_RPW):
                accs[r] = accs[r] + blk[r, pl.ds(j * _L, _L)] * ev
        return accs

    _start(0, xblk0, sem0)

    def pair_body(i, accs):
        accs = list(accs)
        _wait(xblk0, sem0)
        _start(2 * i + 1, xblk1, sem1)
        accs = _accumulate(xblk0, 2 * i * _C, accs)
        _wait(xblk1, sem1)

        @pl.when(i < npairs - 1)
        def _():
            _start(2 * i + 2, xblk0, sem0)

        accs = _accumulate(xblk1, (2 * i + 1) * _C, accs)
        return tuple(accs)

    z = jnp.zeros((_L,), jnp.float32)
    accs = lax.fori_loop(0, npairs, pair_body, (z,) * _RPW)

    yvec = jnp.zeros((_L,), jnp.float32)
    for r in range(_RPW):
        yvec = jnp.where(lane == r,
                         allreduce(accs[r], jnp.add) * inv_total, yvec)
    ybuf[...] = yvec
    pltpu.sync_copy(ybuf, o_hbm.at[pl.ds(wid * _RPW, _RPW)])


def kernel(x, w, k):
    del k
    t, n = x.shape

    y_tc = pl.pallas_call(
        _tc_body,
        grid=(_T_TC // _BT,),
        in_specs=[
            pl.BlockSpec((1, n), lambda i: (0, 0)),
            pl.BlockSpec((_BT, n), lambda i: (i, 0)),
        ],
        out_specs=pl.BlockSpec((_BT, 1), lambda i: (i, 0)),
        out_shape=jax.ShapeDtypeStruct((_T_TC, 1), jnp.float32),
        scratch_shapes=[pltpu.VMEM((1, n), jnp.float32)],
    )(w.reshape(1, n), x)

    y_sc = _sc_matvec(x, w)
    return jnp.concatenate([y_tc.reshape(_T_TC), y_sc])


# speedup vs baseline: 1.6180x; 1.6180x over previous
import functools

import jax
import jax.numpy as jnp
from jax import lax
from jax.experimental import pallas as pl
from jax.experimental.pallas import tpu as pltpu
from jax.experimental.pallas import tpu_sc as plsc

_T, _N = 2048, 32768
_BT = 64
_NW = 32
_T_SC = 256
_RPW = _T_SC // _NW
_T_TC = _T - _T_SC
_L = 16
_U = 8


def _tc_body(w_ref, x_ref, o_ref, sw_ref):
    i = pl.program_id(0)

    @pl.when(i == 0)
    def _():
        wv = w_ref[...]
        m = jnp.max(wv)
        e = jnp.exp(wv - m)
        sw_ref[...] = e / jnp.sum(e)

    o_ref[...] = jnp.sum(x_ref[...] * sw_ref[...], axis=1, keepdims=True)


_sc_mesh = plsc.VectorSubcoreMesh(core_axis_name="c", subcore_axis_name="s")


@functools.partial(
    pl.kernel,
    mesh=_sc_mesh,
    out_type=jax.ShapeDtypeStruct((_NW, _L), jnp.float32),
    scratch_types=[
        pltpu.VMEM((_N,), jnp.float32),
        pltpu.VMEM((_N,), jnp.float32),
        pltpu.VMEM((_N,), jnp.float32),
        pltpu.VMEM((_L,), jnp.float32),
        pltpu.SemaphoreType.DMA,
        pltpu.SemaphoreType.DMA,
    ],
)
def _sc_matvec(x_hbm, w_hbm, o_hbm, ebuf, xrow0, xrow1, ybuf, sem0, sem1):
    wid = lax.axis_index("s") * 2 + lax.axis_index("c")
    nchunks = _N // _L
    lane = lax.iota(jnp.int32, _L)

    def allreduce(v, op):
        for s in (1, 2, 4, 8):
            v = op(v, v.at[lane ^ s].get(mode="promise_in_bounds"))
        return v

    row0 = _T_TC + wid * _RPW
    rowbufs = [(xrow0, sem0), (xrow1, sem1)]

    def _start(r, buf, sem):
        pltpu.make_async_copy(x_hbm.at[row0 + r], buf, sem).start()

    def _wait(buf, sem):
        pltpu.make_async_copy(x_hbm.at[row0], buf, sem).wait()

    _start(0, *rowbufs[0])

    pltpu.sync_copy(w_hbm, ebuf)

    def mx_body(i, accs):
        accs = list(accs)
        base = i * (_U * _L)
        for j in range(_U):
            accs[j % 4] = jnp.maximum(accs[j % 4],
                                      ebuf[pl.ds(base + j * _L, _L)])
        return tuple(accs)

    ninf = jnp.full((_L,), -jnp.inf, jnp.float32)
    m0, m1, m2, m3 = lax.fori_loop(0, nchunks // _U, mx_body, (ninf,) * 4)
    m16 = allreduce(jnp.maximum(jnp.maximum(m0, m1), jnp.maximum(m2, m3)),
                    jnp.maximum)

    def ex_body(i, accs):
        accs = list(accs)
        base = i * (_U * _L)
        for j in range(_U):
            v = jnp.exp(ebuf[pl.ds(base + j * _L, _L)] - m16)
            ebuf[pl.ds(base + j * _L, _L)] = v
            accs[j % 4] = accs[j % 4] + v
        return tuple(accs)

    z = jnp.zeros((_L,), jnp.float32)
    s0, s1, s2, s3 = lax.fori_loop(0, nchunks // _U, ex_body, (z,) * 4)
    inv_total = 1.0 / allreduce((s0 + s1) + (s2 + s3), jnp.add)

    yvec = jnp.zeros((_L,), jnp.float32)
    for r in range(_RPW):
        buf, sem = rowbufs[r % 2]
        _wait(buf, sem)
        if r + 1 < _RPW:
            _start(r + 1, *rowbufs[(r + 1) % 2])

        def dot_body(i, accs, buf=buf):
            accs = list(accs)
            base = i * (_U * _L)
            for j in range(_U):
                o = base + j * _L
                accs[j % 4] = accs[j % 4] + (
                    buf[pl.ds(o, _L)] * ebuf[pl.ds(o, _L)])
            return tuple(accs)

        a0, a1, a2, a3 = lax.fori_loop(0, nchunks // _U, dot_body, (z,) * 4)
        a16 = (a0 + a1) + (a2 + a3)
        yvec = jnp.where(lane == r, allreduce(a16, jnp.add) * inv_total, yvec)

    ybuf[...] = yvec
    pltpu.sync_copy(ybuf, o_hbm.at[wid])


def kernel(x, w, k):
    del k
    t, n = x.shape

    y_tc = pl.pallas_call(
        _tc_body,
        grid=(_T_TC // _BT,),
        in_specs=[
            pl.BlockSpec((1, n), lambda i: (0, 0)),
            pl.BlockSpec((_BT, n), lambda i: (i, 0)),
        ],
        out_specs=pl.BlockSpec((_BT, 1), lambda i: (i, 0)),
        out_shape=jax.ShapeDtypeStruct((_T_TC, 1), jnp.float32),
        scratch_shapes=[pltpu.VMEM((1, n), jnp.float32)],
    )(w.reshape(1, n), x)

    y_sc = _sc_matvec(x, w)[:, :_RPW].reshape(_T_SC)
    return jnp.concatenate([y_tc.reshape(_T_TC), y_sc])


# speedup vs baseline: 2.0545x; 1.2698x over previous
import jax
import jax.numpy as jnp
from jax.experimental import pallas as pl
from jax.experimental.pallas import tpu as pltpu

_BT = 64


def _mv_body(w_ref, x_ref, o_ref, sw_ref):
    i = pl.program_id(0)

    @pl.when(i == 0)
    def _():
        wv = w_ref[...]
        m = jnp.max(wv)
        e = jnp.exp(wv - m)
        sw_ref[...] = e / jnp.sum(e)

    o_ref[...] = jnp.sum(x_ref[...] * sw_ref[...], axis=1, keepdims=True)


def kernel(x, w, k):
    del k
    t, n = x.shape
    bt = min(_BT, t)
    y = pl.pallas_call(
        _mv_body,
        grid=(t // bt,),
        in_specs=[
            pl.BlockSpec((1, n), lambda i: (0, 0)),
            pl.BlockSpec((bt, n), lambda i: (i, 0)),
        ],
        out_specs=pl.BlockSpec((bt, 1), lambda i: (i, 0)),
        out_shape=jax.ShapeDtypeStruct((t, 1), jnp.float32),
        scratch_shapes=[pltpu.VMEM((1, n), jnp.float32)],
    )(w.reshape(1, n), x)
    return y.reshape(t)
